# baseline (speedup 1.0000x reference)
import functools

import jax
import jax.numpy as jnp
from jax import lax
from jax.experimental import pallas as pl
from jax.experimental.pallas import tpu as pltpu
from jax.experimental.pallas import tpu_sc as plsc

_HIDDEN = 768
_GLYPH_DIM = 576
_GL_MAIN = 512
_GL_TAIL = _GLYPH_DIM - _GL_MAIN
_GL_TAILP = 128
_GLYPH_PAD = _GL_MAIN + _GL_TAILP
_PY_OUT = 128
_TAG = 64
_EPS = 1e-12

_N = 8192
_T = 1024
_NW = 32
_BPW = _N // _NW
_CH = 64
_NCHUNK = _BPW // _CH

_TAIL_RB = 1024


def _tail_body(t_ref, o_ref):
    x = t_ref[...]
    lane = lax.broadcasted_iota(jnp.int32, x.shape, 1)
    o_ref[...] = jnp.where(lane < _GL_TAIL, x, 0.0)


def _tail_table(table):
    V = table.shape[0]
    grid = (pl.cdiv(V, _TAIL_RB),)
    return pl.pallas_call(
        _tail_body,
        grid=grid,
        in_specs=[pl.BlockSpec((_TAIL_RB, _GL_TAILP),
                               lambda i: (i, _GL_MAIN // _GL_TAILP))],
        out_specs=pl.BlockSpec((_TAIL_RB, _GL_TAILP), lambda i: (i, 0)),
        out_shape=jax.ShapeDtypeStruct((V, _GL_TAILP), jnp.float32),
    )(table)


def _sc_gather(idx, table, tail):
    mesh = plsc.VectorSubcoreMesh(core_axis_name="c", subcore_axis_name="s")

    @functools.partial(
        pl.kernel,
        mesh=mesh,
        out_type=jax.ShapeDtypeStruct((_N, _GLYPH_PAD), jnp.float32),
        scratch_types=[
            pltpu.VMEM((_NCHUNK, _CH), jnp.int32),
            pltpu.VMEM((2, _CH, _GL_MAIN), jnp.float32),
            pltpu.VMEM((2, _CH, _GL_TAILP), jnp.float32),
            pltpu.SemaphoreType.DMA,
            pltpu.SemaphoreType.DMA,
        ],
    )
    def k(idx_hbm, table_hbm, tail_hbm, out_hbm,
          idx_v, rows_v, trows_v, gsem, tsem):
        wid = lax.axis_index("s") * 2 + lax.axis_index("c")
        pltpu.sync_copy(idx_hbm.at[wid], idx_v)
        gm = pltpu.async_copy(
            table_hbm.at[idx_v.at[0], pl.ds(0, _GL_MAIN)], rows_v.at[0], gsem)
        gt = pltpu.async_copy(tail_hbm.at[idx_v.at[0]], trows_v.at[0], tsem)
        for c in range(_NCHUNK):
            gm.wait()
            gt.wait()
            if c + 1 < _NCHUNK:
                b = (c + 1) % 2
                gm = pltpu.async_copy(
                    table_hbm.at[idx_v.at[c + 1], pl.ds(0, _GL_MAIN)],
                    rows_v.at[b], gsem)
                gt = pltpu.async_copy(
                    tail_hbm.at[idx_v.at[c + 1]], trows_v.at[b], tsem)
            base = wid * _BPW + c * _CH
            pltpu.sync_copy(rows_v.at[c % 2],
                            out_hbm.at[pl.ds(base, _CH), pl.ds(0, _GL_MAIN)])
            pltpu.sync_copy(
                trows_v.at[c % 2],
                out_hbm.at[pl.ds(base, _CH), pl.ds(_GL_MAIN, _GL_TAILP)])

    return k(idx, table, tail)


def _fused_body(word_ref, gl_ref, pin_ref, pos_ref, char_ref, w0_ref, w1_ref,
                cb_ref, tag_ref, fw_ref, fp_ref, fg_ref, ft_ref, fb_ref,
                g_ref, b_ref, out_ref):
    f32 = jnp.float32
    bf16 = jnp.bfloat16
    word = word_ref[...].astype(bf16)
    gl = gl_ref[...].astype(bf16)
    pid = pin_ref[...]
    pos = pos_ref[...]

    c0 = jnp.dot(char_ref[...], w0_ref[...], preferred_element_type=f32)
    c1 = jnp.dot(char_ref[...], w1_ref[...], preferred_element_type=f32)
    c01 = jnp.concatenate([c0, c1], axis=1).astype(bf16)
    e = []
    for t in range(8):
        oh = (pid[:, t:t + 1]
              == lax.broadcasted_iota(jnp.int32, (_T, 32), 1)).astype(bf16)
        e.append(jnp.dot(oh, c01, preferred_element_type=f32))
    py = None
    for t in range(7):
        w = e[t][:, :_PY_OUT] + e[t + 1][:, _PY_OUT:]
        py = w if py is None else jnp.maximum(py, w)
    py = py + cb_ref[...]

    oht = (pos == lax.broadcasted_iota(jnp.int32, (_T, _TAG), 1)).astype(bf16)
    tg = jnp.dot(oht, tag_ref[...], preferred_element_type=f32)

    y = (jnp.dot(word, fw_ref[...], preferred_element_type=f32)
         + jnp.dot(py.astype(bf16), fp_ref[...], preferred_element_type=f32)
         + jnp.dot(gl, fg_ref[...], preferred_element_type=f32)
         + jnp.dot(tg.astype(bf16), ft_ref[...], preferred_element_type=f32)
         + fb_ref[...])

    mu = jnp.mean(y, axis=1, keepdims=True)
    yc = y - mu
    var = jnp.mean(yc * yc, axis=1, keepdims=True)
    out_ref[...] = yc * lax.rsqrt(var + _EPS) * g_ref[...] + b_ref[...]


def _tc_fused(word, rows, pid, pos, char_table, w0T, w1T, conv_b, tag_table,
              fw, fp, fg, ft, fc_b, ln_g, ln_b):
    grid = (_N // _T,)
    full = lambda shape: pl.BlockSpec(shape, lambda i: (0, 0))
    tiled = lambda cols: pl.BlockSpec((_T, cols), lambda i: (i, 0))
    return pl.pallas_call(
        _fused_body,
        grid=grid,
        in_specs=[
            tiled(_HIDDEN),
            tiled(_GLYPH_PAD),
            tiled(8),
            tiled(1),
            full((32, 128)),
            full((128, 128)),
            full((128, 128)),
            full((1, _PY_OUT)),
            full((_TAG, _TAG)),
            full((_HIDDEN, _HIDDEN)),
            full((_PY_OUT, _HIDDEN)),
            full((_GLYPH_PAD, _HIDDEN)),
            full((_TAG, _HIDDEN)),
            full((1, _HIDDEN)),
            full((1, _HIDDEN)),
            full((1, _HIDDEN)),
        ],
        out_specs=tiled(_HIDDEN),
        out_shape=jax.ShapeDtypeStruct((_N, _HIDDEN), jnp.float32),
    )(word, rows, pid, pos, char_table, w0T, w1T, conv_b, tag_table,
      fw, fp, fg, ft, fc_b, ln_g, ln_b)


def kernel(word_embeddings, pinyin_ids, glyph_ids, pos_ids, pos_table,
           glyph_table, pinyin_char_table, pinyin_conv_w, pinyin_conv_b,
           tag_table, fc_w, fc_b, ln_g, ln_b):
    B, S, H = word_embeddings.shape
    word = word_embeddings.reshape(_N, H)
    pid = pinyin_ids.reshape(_N, 8).astype(jnp.int32)
    pos = pos_ids.reshape(_N, 1).astype(jnp.int32)
    gidx = glyph_ids.reshape(_N).astype(jnp.int32).reshape(_NW, _NCHUNK, _CH)

    rows = _sc_gather(gidx, glyph_table, _tail_table(glyph_table))

    bf16 = jnp.bfloat16
    fcT = fc_w.T
    fw = fcT[:H].astype(bf16)
    fp = fcT[H:H + _PY_OUT].astype(bf16)
    gbase = H + _PY_OUT
    fg = jnp.pad(fcT[gbase:gbase + _GLYPH_DIM],
                 ((0, _GLYPH_PAD - _GLYPH_DIM), (0, 0))).astype(bf16)
    ft = fcT[gbase + _GLYPH_DIM:].astype(bf16)
    w0T = pinyin_conv_w[:, :, 0].T
    w1T = pinyin_conv_w[:, :, 1].T

    out = _tc_fused(word, rows, pid, pos, pinyin_char_table, w0T, w1T,
                    pinyin_conv_b.reshape(1, _PY_OUT),
                    tag_table.astype(bf16), fw, fp, fg, ft,
                    fc_b.reshape(1, H), ln_g.reshape(1, H), ln_b.reshape(1, H))
    return out.reshape(B, S, H)


# speedup vs baseline: 1.0022x; 1.0022x over previous
import functools

import jax
import jax.numpy as jnp
from jax import lax
from jax.experimental import pallas as pl
from jax.experimental.pallas import tpu as pltpu
from jax.experimental.pallas import tpu_sc as plsc

_HIDDEN = 768
_GLYPH_DIM = 576
_GL_MAIN = 512
_GL_TAIL = _GLYPH_DIM - _GL_MAIN
_GL_TAILP = 128
_GLYPH_PAD = _GL_MAIN + _GL_TAILP
_PY_OUT = 128
_TAG = 64
_EPS = 1e-12

_N = 8192
_T = 1024
_NW = 32
_BPW = _N // _NW
_CH = 64
_NCHUNK = _BPW // _CH

_TAIL_RB = 1024


def _tail_body(t_ref, o_ref):
    x = t_ref[...]
    lane = lax.broadcasted_iota(jnp.int32, x.shape, 1)
    o_ref[...] = jnp.where(lane < _GL_TAIL, x, 0.0)


def _tail_table(table):
    V = table.shape[0]
    grid = (pl.cdiv(V, _TAIL_RB),)
    return pl.pallas_call(
        _tail_body,
        grid=grid,
        in_specs=[pl.BlockSpec((_TAIL_RB, _GL_TAILP),
                               lambda i: (i, _GL_MAIN // _GL_TAILP))],
        out_specs=pl.BlockSpec((_TAIL_RB, _GL_TAILP), lambda i: (i, 0)),
        out_shape=jax.ShapeDtypeStruct((V, _GL_TAILP), jnp.float32),
    )(table)


def _sc_gather(idx, table, tail):
    mesh = plsc.VectorSubcoreMesh(core_axis_name="c", subcore_axis_name="s")

    @functools.partial(
        pl.kernel,
        mesh=mesh,
        out_type=(
            jax.ShapeDtypeStruct((_N, _GL_MAIN), jnp.float32),
            jax.ShapeDtypeStruct((_N, _GL_TAILP), jnp.float32),
        ),
        scratch_types=[
            pltpu.VMEM((_NCHUNK, _CH), jnp.int32),
            pltpu.VMEM((2, _CH, _GL_MAIN), jnp.float32),
            pltpu.VMEM((2, _CH, _GL_TAILP), jnp.float32),
            pltpu.SemaphoreType.DMA,
            pltpu.SemaphoreType.DMA,
        ],
    )
    def k(idx_hbm, table_hbm, tail_hbm, outm_hbm, outt_hbm,
          idx_v, rows_v, trows_v, gsem, tsem):
        wid = lax.axis_index("s") * 2 + lax.axis_index("c")
        pltpu.sync_copy(idx_hbm.at[wid], idx_v)
        gm = pltpu.async_copy(
            table_hbm.at[idx_v.at[0], pl.ds(0, _GL_MAIN)], rows_v.at[0], gsem)
        gt = pltpu.async_copy(tail_hbm.at[idx_v.at[0]], trows_v.at[0], tsem)
        for c in range(_NCHUNK):
            gm.wait()
            gt.wait()
            if c + 1 < _NCHUNK:
                b = (c + 1) % 2
                gm = pltpu.async_copy(
                    table_hbm.at[idx_v.at[c + 1], pl.ds(0, _GL_MAIN)],
                    rows_v.at[b], gsem)
                gt = pltpu.async_copy(
                    tail_hbm.at[idx_v.at[c + 1]], trows_v.at[b], tsem)
            base = wid * _BPW + c * _CH
            pltpu.sync_copy(rows_v.at[c % 2], outm_hbm.at[pl.ds(base, _CH)])
            pltpu.sync_copy(trows_v.at[c % 2], outt_hbm.at[pl.ds(base, _CH)])

    return k(idx, table, tail)


def _fused_body(word_ref, glm_ref, glt_ref, pin_ref, pos_ref, char_ref, w0_ref, w1_ref,
                cb_ref, tag_ref, fw_ref, fp_ref, fg_ref, ft_ref, fb_ref,
                g_ref, b_ref, out_ref):
    f32 = jnp.float32
    bf16 = jnp.bfloat16
    word = word_ref[...].astype(bf16)
    gl = jnp.concatenate([glm_ref[...], glt_ref[...]],
                         axis=1).astype(bf16)
    pid = pin_ref[...]
    pos = pos_ref[...]

    c0 = jnp.dot(char_ref[...], w0_ref[...], preferred_element_type=f32)
    c1 = jnp.dot(char_ref[...], w1_ref[...], preferred_element_type=f32)
    c01 = jnp.concatenate([c0, c1], axis=1).astype(bf16)
    e = []
    for t in range(8):
        oh = (pid[:, t:t + 1]
              == lax.broadcasted_iota(jnp.int32, (_T, 32), 1)).astype(bf16)
        e.append(jnp.dot(oh, c01, preferred_element_type=f32))
    py = None
    for t in range(7):
        w = e[t][:, :_PY_OUT] + e[t + 1][:, _PY_OUT:]
        py = w if py is None else jnp.maximum(py, w)
    py = py + cb_ref[...]

    oht = (pos == lax.broadcasted_iota(jnp.int32, (_T, _TAG), 1)).astype(bf16)
    tg = jnp.dot(oht, tag_ref[...], preferred_element_type=f32)

    y = (jnp.dot(word, fw_ref[...], preferred_element_type=f32)
         + jnp.dot(py.astype(bf16), fp_ref[...], preferred_element_type=f32)
         + jnp.dot(gl, fg_ref[...], preferred_element_type=f32)
         + jnp.dot(tg.astype(bf16), ft_ref[...], preferred_element_type=f32)
         + fb_ref[...])

    mu = jnp.mean(y, axis=1, keepdims=True)
    yc = y - mu
    var = jnp.mean(yc * yc, axis=1, keepdims=True)
    out_ref[...] = yc * lax.rsqrt(var + _EPS) * g_ref[...] + b_ref[...]


def _tc_fused(word, rowsm, rowst, pid, pos, char_table, w0T, w1T, conv_b, tag_table,
              fw, fp, fg, ft, fc_b, ln_g, ln_b):
    grid = (_N // _T,)
    full = lambda shape: pl.BlockSpec(shape, lambda i: (0, 0))
    tiled = lambda cols: pl.BlockSpec((_T, cols), lambda i: (i, 0))
    return pl.pallas_call(
        _fused_body,
        grid=grid,
        in_specs=[
            tiled(_HIDDEN),
            tiled(_GL_MAIN),
            tiled(_GL_TAILP),
            tiled(8),
            tiled(1),
            full((32, 128)),
            full((128, 128)),
            full((128, 128)),
            full((1, _PY_OUT)),
            full((_TAG, _TAG)),
            full((_HIDDEN, _HIDDEN)),
            full((_PY_OUT, _HIDDEN)),
            full((_GLYPH_PAD, _HIDDEN)),
            full((_TAG, _HIDDEN)),
            full((1, _HIDDEN)),
            full((1, _HIDDEN)),
            full((1, _HIDDEN)),
        ],
        out_specs=tiled(_HIDDEN),
        out_shape=jax.ShapeDtypeStruct((_N, _HIDDEN), jnp.float32),
    )(word, rowsm, rowst, pid, pos, char_table, w0T, w1T, conv_b, tag_table,
      fw, fp, fg, ft, fc_b, ln_g, ln_b)


def kernel(word_embeddings, pinyin_ids, glyph_ids, pos_ids, pos_table,
           glyph_table, pinyin_char_table, pinyin_conv_w, pinyin_conv_b,
           tag_table, fc_w, fc_b, ln_g, ln_b):
    B, S, H = word_embeddings.shape
    word = word_embeddings.reshape(_N, H)
    pid = pinyin_ids.reshape(_N, 8).astype(jnp.int32)
    pos = pos_ids.reshape(_N, 1).astype(jnp.int32)
    gidx = glyph_ids.reshape(_N).astype(jnp.int32).reshape(_NW, _NCHUNK, _CH)

    rowsm, rowst = _sc_gather(gidx, glyph_table, _tail_table(glyph_table))

    bf16 = jnp.bfloat16
    fcT = fc_w.T
    fw = fcT[:H].astype(bf16)
    fp = fcT[H:H + _PY_OUT].astype(bf16)
    gbase = H + _PY_OUT
    fg = jnp.pad(fcT[gbase:gbase + _GLYPH_DIM],
                 ((0, _GLYPH_PAD - _GLYPH_DIM), (0, 0))).astype(bf16)
    ft = fcT[gbase + _GLYPH_DIM:].astype(bf16)
    w0T = pinyin_conv_w[:, :, 0].T
    w1T = pinyin_conv_w[:, :, 1].T

    out = _tc_fused(word, rowsm, rowst, pid, pos, pinyin_char_table, w0T, w1T,
                    pinyin_conv_b.reshape(1, _PY_OUT),
                    tag_table.astype(bf16), fw, fp, fg, ft,
                    fc_b.reshape(1, H), ln_g.reshape(1, H), ln_b.reshape(1, H))
    return out.reshape(B, S, H)
